# baseline (speedup 1.0000x reference)
import functools

import jax
import jax.numpy as jnp
from jax import lax
from jax.experimental import pallas as pl
from jax.experimental.pallas import tpu as pltpu
from jax.experimental.pallas import tpu_sc as plsc

_NUM_CLASSES = 100000
_NUM_FEATURES = 128
_ALPHA = 0.01
_BATCH = 1024
_NC, _NS, _L = 2, 16, 16
_NW = _NC * _NS
_BPW = _BATCH // _NW
_NREG = _NUM_FEATURES // _L
_BN = 5560

_mesh = functools.partial(
    plsc.VectorSubcoreMesh,
    core_axis_name="c", subcore_axis_name="s",
    num_cores=_NC, num_subcores=_NS,
)


def _mm_body(x_ref, m_ref, logits_ref, copy_ref):
    m = m_ref[...]
    logits_ref[...] = lax.dot_general(
        m, x_ref[...], (((1,), (1,)), ((), ())),
        preferred_element_type=jnp.float32)
    copy_ref[...] = m


def _tc_matmul_copy(x, mem):
    logits_t, mem_copy = pl.pallas_call(
        _mm_body,
        grid=(pl.cdiv(_NUM_CLASSES, _BN),),
        in_specs=[
            pl.BlockSpec((_BATCH, _NUM_FEATURES), lambda i: (0, 0)),
            pl.BlockSpec((_BN, _NUM_FEATURES), lambda i: (i, 0)),
        ],
        out_specs=[
            pl.BlockSpec((_BN, _BATCH), lambda i: (i, 0)),
            pl.BlockSpec((_BN, _NUM_FEATURES), lambda i: (i, 0)),
        ],
        out_shape=[
            jax.ShapeDtypeStruct((_NUM_CLASSES, _BATCH), jnp.float32),
            jax.ShapeDtypeStruct((_NUM_CLASSES, _NUM_FEATURES), jnp.float32),
        ],
        compiler_params=pltpu.CompilerParams(
            dimension_semantics=("arbitrary",)),
    )(x, mem)
    return jnp.transpose(logits_t), mem_copy


def _worker_id():
    return lax.axis_index("s") * _NC + lax.axis_index("c")


def _sc_update_body(mem_hbm, x_hbm, tgt_hbm, upd_hbm,
                    t_all, my_t, my_w, mrows, xrows, urows, sem):
    base = _worker_id() * _BPW
    pltpu.sync_copy(tgt_hbm, t_all)
    pltpu.sync_copy(tgt_hbm.at[pl.ds(base, _BPW)], my_t)

    tv = [t_all[pl.ds(base + _L * k, _L)] for k in range(_BPW // _L)]

    def wbody(jc, ws):
        tj_vec = t_all[pl.ds(jc * _L, _L)]
        for e in range(_L):
            tjv = jnp.full((_L,), tj_vec[e], jnp.int32)
            j = jc * _L + e
            ws = tuple(jnp.where(t == tjv, j, w) for t, w in zip(tv, ws))
        return ws

    ws = lax.fori_loop(
        0, _BATCH // _L, wbody,
        tuple(jnp.zeros((_L,), jnp.int32) for _ in tv))
    for k, w in enumerate(ws):
        my_w[pl.ds(_L * k, _L)] = w

    cm = pltpu.async_copy(mem_hbm.at[my_t], mrows, sem)
    cm.wait()
    cx = pltpu.async_copy(x_hbm.at[my_w], xrows, sem)
    cx.wait()

    for r in range(_BPW):
        u = []
        acc = jnp.zeros((_L,), jnp.float32)
        for g in range(_NREG):
            m = mrows[r, pl.ds(_L * g, _L)]
            xx = xrows[r, pl.ds(_L * g, _L)]
            ug = _ALPHA * m + (1.0 - _ALPHA) * xx
            u.append(ug)
            acc = acc + ug * ug
        sv = jnp.full((_L,), jnp.sum(acc), jnp.float32)
        yi = jnp.int32(0x5F3759DF) - (plsc.bitcast(sv, jnp.int32) >> 1)
        y = plsc.bitcast(yi, jnp.float32)
        for _ in range(3):
            y = y * (1.5 - 0.5 * sv * y * y)
        scale = 1.0 / (sv * y + 1e-12)
        for g in range(_NREG):
            urows[r, pl.ds(_L * g, _L)] = u[g] * scale

    pltpu.sync_copy(urows, upd_hbm.at[pl.ds(base, _BPW)])


def _sc_update(mem, x, tgt):
    kern = pl.kernel(
        _sc_update_body,
        out_type=jax.ShapeDtypeStruct((_BATCH, _NUM_FEATURES), jnp.float32),
        mesh=_mesh(),
        compiler_params=pltpu.CompilerParams(needs_layout_passes=False),
        scratch_types=[
            pltpu.VMEM((_BATCH,), jnp.int32),
            pltpu.VMEM((_BPW,), jnp.int32),
            pltpu.VMEM((_BPW,), jnp.int32),
            pltpu.VMEM((_BPW, _NUM_FEATURES), jnp.float32),
            pltpu.VMEM((_BPW, _NUM_FEATURES), jnp.float32),
            pltpu.VMEM((_BPW, _NUM_FEATURES), jnp.float32),
            pltpu.SemaphoreType.DMA,
        ],
    )
    return kern(mem, x, tgt)


def _sc_scatter_body(upd_hbm, tgt_hbm, mem_ref, my_t, rows, sem):
    base = _worker_id() * _BPW
    pltpu.sync_copy(tgt_hbm.at[pl.ds(base, _BPW)], my_t)
    pltpu.sync_copy(upd_hbm.at[pl.ds(base, _BPW)], rows)
    pltpu.async_copy(rows, mem_ref.at[my_t], sem).wait()


def _sc_scatter(upd, tgt, mem_ref):
    kern = pl.kernel(
        _sc_scatter_body,
        out_type=(),
        mesh=_mesh(),
        scratch_types=[
            pltpu.VMEM((_BPW,), jnp.int32),
            pltpu.VMEM((_BPW, _NUM_FEATURES), jnp.float32),
            pltpu.SemaphoreType.DMA,
        ],
    )
    return kern(upd, tgt, mem_ref)


def kernel(inputs, targets, memory):
    targets = targets.astype(jnp.int32)
    logits, mem_copy = _tc_matmul_copy(inputs, memory)
    updated = _sc_update(memory, inputs, targets)
    mem_ref = jax.new_ref(mem_copy)
    _sc_scatter(updated, targets, mem_ref)
    return logits, mem_ref[...]


# speedup vs baseline: 1.0018x; 1.0018x over previous
import functools

import jax
import jax.numpy as jnp
from jax import lax
from jax.experimental import pallas as pl
from jax.experimental.pallas import tpu as pltpu
from jax.experimental.pallas import tpu_sc as plsc

_NUM_CLASSES = 100000
_NUM_FEATURES = 128
_ALPHA = 0.01
_BATCH = 1024
_NC, _NS, _L = 2, 16, 16
_NW = _NC * _NS
_BPW = _BATCH // _NW
_NREG = _NUM_FEATURES // _L
_BN = 5560

_mesh = functools.partial(
    plsc.VectorSubcoreMesh,
    core_axis_name="c", subcore_axis_name="s",
    num_cores=_NC, num_subcores=_NS,
)


def _mm_body(x_ref, m_ref, logits_ref, copy_ref):
    m = m_ref[...]
    logits_ref[...] = lax.dot_general(
        m, x_ref[...], (((1,), (1,)), ((), ())),
        preferred_element_type=jnp.float32)
    copy_ref[...] = m


def _tc_matmul_copy(x, mem):
    logits_t, mem_copy = pl.pallas_call(
        _mm_body,
        grid=(pl.cdiv(_NUM_CLASSES, _BN),),
        in_specs=[
            pl.BlockSpec((_BATCH, _NUM_FEATURES), lambda i: (0, 0)),
            pl.BlockSpec((_BN, _NUM_FEATURES), lambda i: (i, 0)),
        ],
        out_specs=[
            pl.BlockSpec((_BN, _BATCH), lambda i: (i, 0)),
            pl.BlockSpec((_BN, _NUM_FEATURES), lambda i: (i, 0)),
        ],
        out_shape=[
            jax.ShapeDtypeStruct((_NUM_CLASSES, _BATCH), jnp.float32),
            jax.ShapeDtypeStruct((_NUM_CLASSES, _NUM_FEATURES), jnp.float32),
        ],
        compiler_params=pltpu.CompilerParams(
            dimension_semantics=("arbitrary",)),
    )(x, mem)
    return jnp.transpose(logits_t), mem_copy


def _worker_id():
    return lax.axis_index("s") * _NC + lax.axis_index("c")


def _sc_update_body(mem_hbm, x_hbm, tgt_hbm, upd_hbm,
                    t_all, my_t, my_w, mrows, xrows, urows, sem):
    base = _worker_id() * _BPW
    pltpu.sync_copy(tgt_hbm, t_all)
    pltpu.sync_copy(tgt_hbm.at[pl.ds(base, _BPW)], my_t)

    tv = [t_all[pl.ds(base + _L * k, _L)] for k in range(_BPW // _L)]

    def wbody(jc, ws):
        tj_vec = t_all[pl.ds(jc * _L, _L)]
        for e in range(_L):
            tjv = jnp.full((_L,), tj_vec[e], jnp.int32)
            j = jc * _L + e
            ws = tuple(jnp.where(t == tjv, j, w) for t, w in zip(tv, ws))
        return ws

    ws = lax.fori_loop(
        0, _BATCH // _L, wbody,
        tuple(jnp.zeros((_L,), jnp.int32) for _ in tv))
    for k, w in enumerate(ws):
        my_w[pl.ds(_L * k, _L)] = w

    cm = pltpu.async_copy(mem_hbm.at[my_t], mrows, sem)
    cm.wait()
    cx = pltpu.async_copy(x_hbm.at[my_w], xrows, sem)
    cx.wait()

    def rbody(r, carry):
        u = []
        acc = jnp.zeros((_L,), jnp.float32)
        for g in range(_NREG):
            m = mrows[r, pl.ds(_L * g, _L)]
            xx = xrows[r, pl.ds(_L * g, _L)]
            ug = _ALPHA * m + (1.0 - _ALPHA) * xx
            u.append(ug)
            acc = acc + ug * ug
        sv = jnp.full((_L,), jnp.sum(acc), jnp.float32)
        yi = jnp.int32(0x5F3759DF) - (plsc.bitcast(sv, jnp.int32) >> 1)
        y = plsc.bitcast(yi, jnp.float32)
        for _ in range(3):
            y = y * (1.5 - 0.5 * sv * y * y)
        scale = 1.0 / (sv * y + 1e-12)
        for g in range(_NREG):
            urows[r, pl.ds(_L * g, _L)] = u[g] * scale
        return carry

    lax.fori_loop(0, _BPW, rbody, jnp.int32(0))

    pltpu.sync_copy(urows, upd_hbm.at[pl.ds(base, _BPW)])


def _sc_update(mem, x, tgt):
    kern = pl.kernel(
        _sc_update_body,
        out_type=jax.ShapeDtypeStruct((_BATCH, _NUM_FEATURES), jnp.float32),
        mesh=_mesh(),
        compiler_params=pltpu.CompilerParams(needs_layout_passes=False),
        scratch_types=[
            pltpu.VMEM((_BATCH,), jnp.int32),
            pltpu.VMEM((_BPW,), jnp.int32),
            pltpu.VMEM((_BPW,), jnp.int32),
            pltpu.VMEM((_BPW, _NUM_FEATURES), jnp.float32),
            pltpu.VMEM((_BPW, _NUM_FEATURES), jnp.float32),
            pltpu.VMEM((_BPW, _NUM_FEATURES), jnp.float32),
            pltpu.SemaphoreType.DMA,
        ],
    )
    return kern(mem, x, tgt)


def _sc_scatter_body(upd_hbm, tgt_hbm, mem_ref, my_t, rows, sem):
    base = _worker_id() * _BPW
    pltpu.sync_copy(tgt_hbm.at[pl.ds(base, _BPW)], my_t)
    pltpu.sync_copy(upd_hbm.at[pl.ds(base, _BPW)], rows)
    pltpu.async_copy(rows, mem_ref.at[my_t], sem).wait()


def _sc_scatter(upd, tgt, mem_ref):
    kern = pl.kernel(
        _sc_scatter_body,
        out_type=(),
        mesh=_mesh(),
        scratch_types=[
            pltpu.VMEM((_BPW,), jnp.int32),
            pltpu.VMEM((_BPW, _NUM_FEATURES), jnp.float32),
            pltpu.SemaphoreType.DMA,
        ],
    )
    return kern(upd, tgt, mem_ref)


def kernel(inputs, targets, memory):
    targets = targets.astype(jnp.int32)
    logits, mem_copy = _tc_matmul_copy(inputs, memory)
    updated = _sc_update(memory, inputs, targets)
    mem_ref = jax.new_ref(mem_copy)
    _sc_scatter(updated, targets, mem_ref)
    return logits, mem_ref[...]


# speedup vs baseline: 1.0034x; 1.0017x over previous
import functools

import jax
import jax.numpy as jnp
from jax import lax
from jax.experimental import pallas as pl
from jax.experimental.pallas import tpu as pltpu
from jax.experimental.pallas import tpu_sc as plsc

_NUM_CLASSES = 100000
_NUM_FEATURES = 128
_ALPHA = 0.01
_BATCH = 1024
_NC, _NS, _L = 2, 16, 16
_NW = _NC * _NS
_BPW = _BATCH // _NW
_NREG = _NUM_FEATURES // _L
_BN = 5560

_mesh = functools.partial(
    plsc.VectorSubcoreMesh,
    core_axis_name="c", subcore_axis_name="s",
    num_cores=_NC, num_subcores=_NS,
)


def _mm_body(x_ref, m_ref, logits_ref, copy_ref):
    m = m_ref[...]
    logits_ref[...] = lax.dot_general(
        m, x_ref[...], (((1,), (1,)), ((), ())),
        preferred_element_type=jnp.float32)
    copy_ref[...] = m


def _tc_matmul_copy(x, mem):
    logits_t, mem_copy = pl.pallas_call(
        _mm_body,
        grid=(pl.cdiv(_NUM_CLASSES, _BN),),
        in_specs=[
            pl.BlockSpec((_BATCH, _NUM_FEATURES), lambda i: (0, 0)),
            pl.BlockSpec((_BN, _NUM_FEATURES), lambda i: (i, 0)),
        ],
        out_specs=[
            pl.BlockSpec((_BN, _BATCH), lambda i: (i, 0)),
            pl.BlockSpec((_BN, _NUM_FEATURES), lambda i: (i, 0)),
        ],
        out_shape=[
            jax.ShapeDtypeStruct((_NUM_CLASSES, _BATCH), jnp.float32),
            jax.ShapeDtypeStruct((_NUM_CLASSES, _NUM_FEATURES), jnp.float32),
        ],
        compiler_params=pltpu.CompilerParams(
            dimension_semantics=("arbitrary",)),
    )(x, mem)
    return jnp.transpose(logits_t), mem_copy


def _worker_id():
    return lax.axis_index("s") * _NC + lax.axis_index("c")


def _sc_update_body(mem_hbm, x_hbm, tgt_hbm, upd_hbm,
                    t_all, my_t, my_w, mrows, xrows, urows, sem):
    base = _worker_id() * _BPW
    pltpu.sync_copy(tgt_hbm, t_all)
    pltpu.sync_copy(tgt_hbm.at[pl.ds(base, _BPW)], my_t)

    tv = [t_all[pl.ds(base + _L * k, _L)] for k in range(_BPW // _L)]

    def wbody(jc, ws):
        tj_vec = t_all[pl.ds(jc * _L, _L)]
        for e in range(_L):
            tjv = jnp.full((_L,), tj_vec[e], jnp.int32)
            j = jc * _L + e
            ws = tuple(jnp.where(t == tjv, j, w) for t, w in zip(tv, ws))
        return ws

    ws = lax.fori_loop(
        0, _BATCH // _L, wbody,
        tuple(jnp.zeros((_L,), jnp.int32) for _ in tv))
    for k, w in enumerate(ws):
        my_w[pl.ds(_L * k, _L)] = w

    cm = pltpu.async_copy(mem_hbm.at[my_t], mrows, sem)
    cm.wait()
    cx = pltpu.async_copy(x_hbm.at[my_w], xrows, sem)
    cx.wait()

    def rbody(r, carry):
        u = []
        acc = jnp.zeros((_L,), jnp.float32)
        for g in range(_NREG):
            m = mrows[r, pl.ds(_L * g, _L)]
            xx = xrows[r, pl.ds(_L * g, _L)]
            ug = _ALPHA * m + (1.0 - _ALPHA) * xx
            u.append(ug)
            acc = acc + ug * ug
        sv = jnp.full((_L,), jnp.sum(acc), jnp.float32)
        yi = jnp.int32(0x5F3759DF) - (plsc.bitcast(sv, jnp.int32) >> 1)
        y = plsc.bitcast(yi, jnp.float32)
        for _ in range(3):
            y = y * (1.5 - 0.5 * sv * y * y)
        scale = 1.0 / (sv * y + 1e-12)
        for g in range(_NREG):
            urows[r, pl.ds(_L * g, _L)] = u[g] * scale
        return carry

    lax.fori_loop(0, _BPW, rbody, jnp.int32(0))

    pltpu.sync_copy(urows, upd_hbm.at[pl.ds(base, _BPW)])


def _sc_update(mem, x, tgt):
    kern = pl.kernel(
        _sc_update_body,
        out_type=jax.ShapeDtypeStruct((_BATCH, _NUM_FEATURES), jnp.float32),
        mesh=_mesh(),
        compiler_params=pltpu.CompilerParams(needs_layout_passes=False),
        scratch_types=[
            pltpu.VMEM((_BATCH,), jnp.int32),
            pltpu.VMEM((_BPW,), jnp.int32),
            pltpu.VMEM((_BPW,), jnp.int32),
            pltpu.VMEM((_BPW, _NUM_FEATURES), jnp.float32),
            pltpu.VMEM((_BPW, _NUM_FEATURES), jnp.float32),
            pltpu.VMEM((_BPW, _NUM_FEATURES), jnp.float32),
            pltpu.SemaphoreType.DMA,
        ],
    )
    return kern(mem, x, tgt)


def _sc_scatter_body(upd_hbm, tgt_hbm, mem_ref, my_t, rows, sem, sem2):
    base = _worker_id() * _BPW
    ct = pltpu.async_copy(tgt_hbm.at[pl.ds(base, _BPW)], my_t, sem2)
    cr = pltpu.async_copy(upd_hbm.at[pl.ds(base, _BPW)], rows, sem)
    ct.wait()
    cr.wait()
    pltpu.async_copy(rows, mem_ref.at[my_t], sem).wait()


def _sc_scatter(upd, tgt, mem_ref):
    kern = pl.kernel(
        _sc_scatter_body,
        out_type=(),
        mesh=_mesh(),
        scratch_types=[
            pltpu.VMEM((_BPW,), jnp.int32),
            pltpu.VMEM((_BPW, _NUM_FEATURES), jnp.float32),
            pltpu.SemaphoreType.DMA,
            pltpu.SemaphoreType.DMA,
        ],
    )
    return kern(upd, tgt, mem_ref)


def kernel(inputs, targets, memory):
    targets = targets.astype(jnp.int32)
    logits, mem_copy = _tc_matmul_copy(inputs, memory)
    updated = _sc_update(memory, inputs, targets)
    mem_ref = jax.new_ref(mem_copy)
    _sc_scatter(updated, targets, mem_ref)
    return logits, mem_ref[...]
